# baseline (speedup 1.0000x reference)
import jax
import jax.numpy as jnp
from jax import lax
from jax.experimental import pallas as pl
from jax.experimental.pallas import tpu as pltpu
from jax.experimental.pallas import tpu_sc as plsc

D_MODEL = 128
MAX_REL = 128
SEQ = 1024
RPAD = 2 * SEQ
NC, NS, L = 2, 16, 16
NW = NC * NS
FILL = RPAD // NS
ROWS_PER_W = SEQ // NW
HALF = SEQ // 2
WIN = HALF + ROWS_PER_W - 1
NBUF = 8
EMBV = FILL + 8
EPAD = 264


def _rel_pos_body(emb_hbm, out_hbm, emb_v, rows_v, win_v, r_sh, dsem, ssem):
    c = lax.axis_index("c")
    s = lax.axis_index("s")

    base = s * FILL
    src_min = jnp.clip((SEQ - 1) - (base + FILL - 1), -MAX_REL, MAX_REL) + MAX_REL
    start = jnp.minimum((src_min // 8) * 8, MAX_REL)
    pltpu.sync_copy(emb_hbm.at[pl.ds(start, EMBV)], emb_v)

    def fill_row(t, _):
        src = jnp.clip((SEQ - 1) - (base + t), -MAX_REL, MAX_REL) + MAX_REL
        for k in range(D_MODEL // L):
            rows_v[t, pl.ds(k * L, L)] = emb_v[src - start, pl.ds(k * L, L)]
        return 0

    lax.fori_loop(0, FILL, fill_row, 0)
    pltpu.sync_copy(rows_v, r_sh.at[pl.ds(base, FILL)])
    plsc.subcore_barrier()

    w = s * NC + c
    i0 = w * ROWS_PER_W

    pending = []

    def fire_second_half(i):
        pending.append(
            pltpu.async_copy(
                r_sh.at[pl.ds((SEQ - 1) - i + HALF, HALF)],
                out_hbm.at[i, pl.ds(HALF, HALF)],
                dsem,
            )
        )
        if len(pending) >= NBUF:
            pending.pop(0).wait()

    for r in range(NBUF):
        fire_second_half(i0 + r)

    wbase = (SEQ - ROWS_PER_W) - i0
    pltpu.sync_copy(r_sh.at[pl.ds(wbase, WIN)], win_v)
    streams = [
        pltpu.async_copy(
            win_v.at[pl.ds((ROWS_PER_W - 1) - r, HALF)],
            out_hbm.at[i0 + r, pl.ds(0, HALF)],
            ssem,
        )
        for r in range(ROWS_PER_W)
    ]

    for r in range(NBUF, ROWS_PER_W):
        fire_second_half(i0 + r)
    for d in pending:
        d.wait()
    for d in streams:
        d.wait()


@jax.jit
def _rel_pos_sc(embeddings):
    mesh = plsc.VectorSubcoreMesh(
        core_axis_name="c", subcore_axis_name="s",
        num_cores=NC, num_subcores=NS,
    )
    return pl.kernel(
        _rel_pos_body,
        out_type=jax.ShapeDtypeStruct((SEQ, SEQ, D_MODEL), jnp.float32),
        mesh=mesh,
        scratch_types=[
            pltpu.VMEM((EMBV, D_MODEL), jnp.float32),
            pltpu.VMEM((FILL, D_MODEL), jnp.float32),
            pltpu.VMEM((WIN, D_MODEL), jnp.float32),
            pltpu.VMEM_SHARED((RPAD, D_MODEL), jnp.float32),
            pltpu.SemaphoreType.DMA,
            pltpu.SemaphoreType.DMA,
        ],
    )(embeddings)


def kernel(embeddings, seq_len):
    del seq_len
    emb_pad = jnp.pad(embeddings, ((0, EPAD - embeddings.shape[0]), (0, 0)))
    return _rel_pos_sc(emb_pad)


# speedup vs baseline: 1.0038x; 1.0038x over previous
import jax
import jax.numpy as jnp
from jax import lax
from jax.experimental import pallas as pl
from jax.experimental.pallas import tpu as pltpu
from jax.experimental.pallas import tpu_sc as plsc

D_MODEL = 128
MAX_REL = 128
SEQ = 1024
RPAD = 2 * SEQ
NC, NS, L = 2, 16, 16
NW = NC * NS
FILL = RPAD // NS
ROWS_PER_W = SEQ // NW
HALF = SEQ // 2
CUT = 576
WIN = CUT + ROWS_PER_W - 1
NBUF = 8
EMBV = FILL + 8
EPAD = 264


def _rel_pos_body(emb_hbm, out_hbm, emb_v, rows_v, win_v, r_sh, dsem, ssem):
    c = lax.axis_index("c")
    s = lax.axis_index("s")

    base = s * FILL
    src_min = jnp.clip((SEQ - 1) - (base + FILL - 1), -MAX_REL, MAX_REL) + MAX_REL
    start = jnp.minimum((src_min // 8) * 8, MAX_REL)
    pltpu.sync_copy(emb_hbm.at[pl.ds(start, EMBV)], emb_v)

    def fill_row(t, _):
        src = jnp.clip((SEQ - 1) - (base + t), -MAX_REL, MAX_REL) + MAX_REL
        for k in range(D_MODEL // L):
            rows_v[t, pl.ds(k * L, L)] = emb_v[src - start, pl.ds(k * L, L)]
        return 0

    lax.fori_loop(0, FILL, fill_row, 0)
    pltpu.sync_copy(rows_v, r_sh.at[pl.ds(base, FILL)])
    plsc.subcore_barrier()

    w = s * NC + c
    i0 = w * ROWS_PER_W

    pending = []

    def fire_second_half(i):
        pending.append(
            pltpu.async_copy(
                r_sh.at[pl.ds((SEQ - 1) - i + CUT, SEQ - CUT)],
                out_hbm.at[i, pl.ds(CUT, SEQ - CUT)],
                dsem,
            )
        )
        if len(pending) >= NBUF:
            pending.pop(0).wait()

    for r in range(NBUF):
        fire_second_half(i0 + r)

    wbase = (SEQ - ROWS_PER_W) - i0
    pltpu.sync_copy(r_sh.at[pl.ds(wbase, WIN)], win_v)
    streams = [
        pltpu.async_copy(
            win_v.at[pl.ds((ROWS_PER_W - 1) - r, CUT)],
            out_hbm.at[i0 + r, pl.ds(0, CUT)],
            ssem,
        )
        for r in range(ROWS_PER_W)
    ]

    for r in range(NBUF, ROWS_PER_W):
        fire_second_half(i0 + r)
    for d in pending:
        d.wait()
    for d in streams:
        d.wait()


@jax.jit
def _rel_pos_sc(embeddings):
    mesh = plsc.VectorSubcoreMesh(
        core_axis_name="c", subcore_axis_name="s",
        num_cores=NC, num_subcores=NS,
    )
    return pl.kernel(
        _rel_pos_body,
        out_type=jax.ShapeDtypeStruct((SEQ, SEQ, D_MODEL), jnp.float32),
        mesh=mesh,
        scratch_types=[
            pltpu.VMEM((EMBV, D_MODEL), jnp.float32),
            pltpu.VMEM((FILL, D_MODEL), jnp.float32),
            pltpu.VMEM((WIN, D_MODEL), jnp.float32),
            pltpu.VMEM_SHARED((RPAD, D_MODEL), jnp.float32),
            pltpu.SemaphoreType.DMA,
            pltpu.SemaphoreType.DMA,
        ],
    )(embeddings)


def kernel(embeddings, seq_len):
    del seq_len
    emb_pad = jnp.pad(embeddings, ((0, EPAD - embeddings.shape[0]), (0, 0)))
    return _rel_pos_sc(emb_pad)


# speedup vs baseline: 1.0046x; 1.0008x over previous
import jax
import jax.numpy as jnp
from jax import lax
from jax.experimental import pallas as pl
from jax.experimental.pallas import tpu as pltpu
from jax.experimental.pallas import tpu_sc as plsc

D_MODEL = 128
MAX_REL = 128
SEQ = 1024
RPAD = 2 * SEQ
NC, NS, L = 2, 16, 16
NW = NC * NS
FILL = RPAD // NS
ROWS_PER_W = SEQ // NW
HALF = SEQ // 2
CUT = 600
WIN = CUT + ROWS_PER_W - 1
NBUF = 8
EMBV = FILL + 8
EPAD = 264


def _rel_pos_body(emb_hbm, out_hbm, emb_v, rows_v, win_v, r_sh, dsem, ssem):
    c = lax.axis_index("c")
    s = lax.axis_index("s")

    base = s * FILL
    src_min = jnp.clip((SEQ - 1) - (base + FILL - 1), -MAX_REL, MAX_REL) + MAX_REL
    start = jnp.minimum((src_min // 8) * 8, MAX_REL)
    pltpu.sync_copy(emb_hbm.at[pl.ds(start, EMBV)], emb_v)

    def fill_row(t, _):
        src = jnp.clip((SEQ - 1) - (base + t), -MAX_REL, MAX_REL) + MAX_REL
        for k in range(D_MODEL // L):
            rows_v[t, pl.ds(k * L, L)] = emb_v[src - start, pl.ds(k * L, L)]
        return 0

    lax.fori_loop(0, FILL, fill_row, 0)
    pltpu.sync_copy(rows_v, r_sh.at[pl.ds(base, FILL)])
    plsc.subcore_barrier()

    w = s * NC + c
    i0 = w * ROWS_PER_W

    pending = []

    def fire_second_half(i):
        pending.append(
            pltpu.async_copy(
                r_sh.at[pl.ds((SEQ - 1) - i + CUT, SEQ - CUT)],
                out_hbm.at[i, pl.ds(CUT, SEQ - CUT)],
                dsem,
            )
        )
        if len(pending) >= NBUF:
            pending.pop(0).wait()

    for r in range(NBUF):
        fire_second_half(i0 + r)

    wbase = (SEQ - ROWS_PER_W) - i0
    pltpu.sync_copy(r_sh.at[pl.ds(wbase, WIN)], win_v)
    streams = [
        pltpu.async_copy(
            win_v.at[pl.ds((ROWS_PER_W - 1) - r, CUT)],
            out_hbm.at[i0 + r, pl.ds(0, CUT)],
            ssem,
        )
        for r in range(ROWS_PER_W)
    ]

    for r in range(NBUF, ROWS_PER_W):
        fire_second_half(i0 + r)
    for d in pending:
        d.wait()
    for d in streams:
        d.wait()


@jax.jit
def _rel_pos_sc(embeddings):
    mesh = plsc.VectorSubcoreMesh(
        core_axis_name="c", subcore_axis_name="s",
        num_cores=NC, num_subcores=NS,
    )
    return pl.kernel(
        _rel_pos_body,
        out_type=jax.ShapeDtypeStruct((SEQ, SEQ, D_MODEL), jnp.float32),
        mesh=mesh,
        scratch_types=[
            pltpu.VMEM((EMBV, D_MODEL), jnp.float32),
            pltpu.VMEM((FILL, D_MODEL), jnp.float32),
            pltpu.VMEM((WIN, D_MODEL), jnp.float32),
            pltpu.VMEM_SHARED((RPAD, D_MODEL), jnp.float32),
            pltpu.SemaphoreType.DMA,
            pltpu.SemaphoreType.DMA,
        ],
    )(embeddings)


def kernel(embeddings, seq_len):
    del seq_len
    emb_pad = jnp.pad(embeddings, ((0, EPAD - embeddings.shape[0]), (0, 0)))
    return _rel_pos_sc(emb_pad)
